# baseline (speedup 1.0000x reference)
import jax
import jax.numpy as jnp
from jax.experimental import pallas as pl

_SZ = 4096
_TEMP = 10.0
_BM = 512
_BN = 512


def _gumbel_pair_kernel(g_ref, u_ref, o_ref):
    g = g_ref[...]
    u = u_ref[...]
    z = g + (-jnp.log(-jnp.log(u + 1e-20) + 1e-20))
    k = jax.lax.broadcasted_iota(jnp.int32, (_BN, _BN // 2), 0)
    j = jax.lax.broadcasted_iota(jnp.int32, (_BN, _BN // 2), 1)
    sel = jnp.where(k == 2 * j, 1.0, jnp.where(k == 2 * j + 1, -1.0, 0.0))
    d = jax.lax.dot_general(
        z, sel,
        dimension_numbers=(((1,), (0,)), ((), ())),
        preferred_element_type=jnp.float32,
    )
    o_ref[...] = jax.nn.sigmoid(d * (1.0 / _TEMP))


def kernel(gen_matrix, uniform_noise):
    g2 = gen_matrix.reshape(_SZ, 2 * _SZ)
    u2 = uniform_noise.reshape(_SZ, 2 * _SZ)
    grid = (_SZ // _BM, 2 * _SZ // _BN)
    return pl.pallas_call(
        _gumbel_pair_kernel,
        grid=grid,
        in_specs=[
            pl.BlockSpec((_BM, _BN), lambda i, j: (i, j)),
            pl.BlockSpec((_BM, _BN), lambda i, j: (i, j)),
        ],
        out_specs=pl.BlockSpec((_BM, _BN // 2), lambda i, j: (i, j)),
        out_shape=jax.ShapeDtypeStruct((_SZ, _SZ), jnp.float32),
    )(g2, u2)


# speedup vs baseline: 61.1737x; 61.1737x over previous
import jax
import jax.numpy as jnp
from jax.experimental import pallas as pl

_SZ = 4096
_TEMP = 10.0
_BR = 128


def _gumbel_pair_kernel(g_ref, u_ref, o_ref):
    g = g_ref[...]
    u = u_ref[...]
    z = g + (-jnp.log(-jnp.log(u + 1e-20) + 1e-20))
    z3 = z.reshape(z.shape[0] // 2, 2, z.shape[1])
    d = z3[:, 0, :] - z3[:, 1, :]
    o_ref[...] = jax.nn.sigmoid(d * (1.0 / _TEMP))


def kernel(gen_matrix, uniform_noise):
    gt = gen_matrix.transpose(0, 2, 1).reshape(2 * _SZ, _SZ)
    ut = uniform_noise.reshape(_SZ, _SZ, 2).transpose(0, 2, 1).reshape(2 * _SZ, _SZ)
    grid = (_SZ // _BR,)
    return pl.pallas_call(
        _gumbel_pair_kernel,
        grid=grid,
        in_specs=[
            pl.BlockSpec((2 * _BR, _SZ), lambda i: (i, 0)),
            pl.BlockSpec((2 * _BR, _SZ), lambda i: (i, 0)),
        ],
        out_specs=pl.BlockSpec((_BR, _SZ), lambda i: (i, 0)),
        out_shape=jax.ShapeDtypeStruct((_SZ, _SZ), jnp.float32),
    )(gt, ut)


# speedup vs baseline: 108.2148x; 1.7690x over previous
import jax
import jax.numpy as jnp
from jax.experimental import pallas as pl

_SZ = 4096
_TEMP = 10.0
_R = 128


def _gumbel_pair_kernel(g_ref, u_ref, o_ref):
    g = g_ref[...]
    u = u_ref[...]
    z = g + (-jnp.log(-jnp.log(u + 1e-20) + 1e-20))
    z3 = z.reshape(z.shape[0] // 2, 2, 128)
    d = z3[:, 0, :] - z3[:, 1, :]
    s = jax.nn.sigmoid(d * (1.0 / _TEMP))
    o_ref[...] = s.reshape(_R, _SZ)


def _native_view(x):
    return x.reshape(_SZ, 32, 128, 2).transpose(0, 1, 3, 2).reshape(64 * _SZ, 128)


def kernel(gen_matrix, uniform_noise):
    gt = _native_view(gen_matrix)
    ut = _native_view(uniform_noise)
    grid = (_SZ // _R,)
    return pl.pallas_call(
        _gumbel_pair_kernel,
        grid=grid,
        in_specs=[
            pl.BlockSpec((64 * _R, 128), lambda i: (i, 0)),
            pl.BlockSpec((64 * _R, 128), lambda i: (i, 0)),
        ],
        out_specs=pl.BlockSpec((_R, _SZ), lambda i: (i, 0)),
        out_shape=jax.ShapeDtypeStruct((_SZ, _SZ), jnp.float32),
    )(gt, ut)
